# baseline (speedup 1.0000x reference)
import jax
import jax.numpy as jnp
from jax import lax
from jax.experimental import pallas as pl
from jax.experimental.pallas import tpu as pltpu
from jax.experimental.pallas import tpu_sc as plsc

N = 10000
NPAD = 10240
E = 320000
D_IN = 128
D_H = 64
D_OUT = 64
NC = 2
NS = 16
NT = NC * NS
CB = 80
NCH = 125
EPT = CB * NCH
E_PAD = NT * EPT
DW = 16
RPT = NPAD // NS
NBUF = 4
BM = 2000

_mesh = plsc.VectorSubcoreMesh(core_axis_name="c", subcore_axis_name="s",
                               num_cores=NC, num_subcores=NS)
_sc_params = pltpu.CompilerParams(use_tc_tiling_on_sc=False)


def _deg_body(edge_hbm, ones_hbm, zeros_hbm, out_hbm, idx_v, ones_v, acc_sh, sem):
    c = lax.axis_index("c")
    s = lax.axis_index("s")
    wid = s * NC + c
    pltpu.sync_copy(zeros_hbm.at[pl.ds(s * RPT, RPT)], acc_sh.at[pl.ds(s * RPT, RPT)])
    pltpu.sync_copy(edge_hbm.at[1, wid], idx_v)
    pltpu.sync_copy(ones_hbm, ones_v)
    plsc.subcore_barrier()

    @pl.loop(0, NCH)
    def _(j):
        pltpu.sync_copy(ones_v, acc_sh.at[idx_v.at[j]], add=True)

    plsc.subcore_barrier()
    pltpu.sync_copy(acc_sh.at[pl.ds(s * RPT, RPT)],
                    out_hbm.at[c, pl.ds(s * RPT, RPT)])


def _sc_degree(edge4, ones, zeros16):
    f = pl.kernel(
        _deg_body,
        out_type=jax.ShapeDtypeStruct((NC, NPAD, DW), jnp.float32),
        mesh=_mesh,
        scratch_types=[
            pltpu.VMEM((NCH, CB), jnp.int32),
            pltpu.VMEM((CB, DW), jnp.float32),
            pltpu.VMEM_SHARED((NPAD, DW), jnp.float32),
            pltpu.SemaphoreType.DMA,
        ],
        compiler_params=_sc_params,
    )
    return f(edge4, ones, zeros16)


def _scat_body(edge_hbm, g_hbm, zeros_hbm, out_hbm,
               si_v, di_v, bufs, acc_sh, gsems):
    c = lax.axis_index("c")
    s = lax.axis_index("s")
    wid = s * NC + c
    pltpu.sync_copy(zeros_hbm.at[pl.ds(s * RPT, RPT)], acc_sh.at[pl.ds(s * RPT, RPT)])
    pltpu.sync_copy(edge_hbm.at[0, wid], si_v)
    pltpu.sync_copy(edge_hbm.at[1, wid], di_v)
    plsc.subcore_barrier()

    def fire_gather(j, r):
        pltpu.async_copy(g_hbm.at[si_v.at[j]], bufs[r], gsems[r])

    def wait_gather(r):
        pltpu.make_async_copy(g_hbm.at[si_v.at[0]], bufs[r], gsems[r]).wait()

    for k in range(NBUF):
        fire_gather(k, k)

    steady = NCH - NBUF
    @pl.loop(0, steady // NBUF)
    def _(i):
        for k in range(NBUF):
            t = i * NBUF + k
            wait_gather(k)
            pltpu.sync_copy(bufs[k], acc_sh.at[di_v.at[t]], add=True)
            fire_gather(t + NBUF, k)

    for t in range(NBUF * (steady // NBUF), NCH):
        k = t % NBUF
        wait_gather(k)
        pltpu.sync_copy(bufs[k], acc_sh.at[di_v.at[t]], add=True)
        if t + NBUF < NCH:
            fire_gather(t + NBUF, k)

    plsc.subcore_barrier()
    pltpu.sync_copy(acc_sh.at[pl.ds(s * RPT, RPT)],
                    out_hbm.at[c, pl.ds(s * RPT, RPT)])


def _sc_aggregate(edge4, g, zeros64):
    f = pl.kernel(
        _scat_body,
        out_type=jax.ShapeDtypeStruct((NC, NPAD, D_H), jnp.float32),
        mesh=_mesh,
        scratch_types=[
            pltpu.VMEM((NCH, CB), jnp.int32),
            pltpu.VMEM((NCH, CB), jnp.int32),
            [pltpu.VMEM((CB, D_H), jnp.float32) for _ in range(NBUF)],
            pltpu.VMEM_SHARED((NPAD, D_H), jnp.float32),
            [pltpu.SemaphoreType.DMA for _ in range(NBUF)],
        ],
        compiler_params=_sc_params,
    )
    return f(edge4, g, zeros64)


def _dis_block(dga_ref, dgb_ref):
    deg = jnp.sum(dga_ref[0] + dgb_ref[0], axis=1, keepdims=True) * (1.0 / DW) + 1.0
    return 1.0 / jnp.sqrt(deg)


def _mm_body(x_ref, w_ref, h_ref):
    h_ref[...] = jnp.dot(x_ref[...], w_ref[...], preferred_element_type=jnp.float32)


def _pre_body(h_ref, dga_ref, dgb_ref, g_ref):
    dis = _dis_block(dga_ref, dgb_ref)
    g_ref[...] = h_ref[...] * dis


def _mid_body(aa_ref, ab_ref, g_ref, w_ref, b_ref, dga_ref, dgb_ref, o_ref):
    dis = _dis_block(dga_ref, dgb_ref)
    act = dis * (aa_ref[0] + ab_ref[0] + g_ref[...]) + b_ref[...]
    act = jnp.maximum(act, 0.0)
    h = jnp.dot(act, w_ref[...], preferred_element_type=jnp.float32)
    o_ref[...] = h * dis


def _fin_body(aa_ref, ab_ref, g_ref, b_ref, dga_ref, dgb_ref, o_ref):
    dis = _dis_block(dga_ref, dgb_ref)
    o_ref[...] = dis * (aa_ref[0] + ab_ref[0] + g_ref[...]) + b_ref[...]


def _row_spec(w):
    return pl.BlockSpec((BM, w), lambda i: (i, 0))


def _core_spec(w, core):
    return pl.BlockSpec((1, BM, w), lambda i, _c=core: (_c, i, 0))


def _full_spec(h, w):
    return pl.BlockSpec((h, w), lambda i: (0, 0))


def _tc_mm(x, w0):
    return pl.pallas_call(
        _mm_body,
        grid=(N // BM,),
        in_specs=[_row_spec(D_IN), _full_spec(D_IN, D_H)],
        out_specs=_row_spec(D_H),
        out_shape=jax.ShapeDtypeStruct((N, D_H), jnp.float32),
    )(x, w0)


def _tc_pre(h, degp):
    return pl.pallas_call(
        _pre_body,
        grid=(N // BM,),
        in_specs=[_row_spec(D_H), _core_spec(DW, 0), _core_spec(DW, 1)],
        out_specs=_row_spec(D_H),
        out_shape=jax.ShapeDtypeStruct((N, D_H), jnp.float32),
    )(h, degp, degp)


def _tc_mid(a, g, w, b, degp):
    return pl.pallas_call(
        _mid_body,
        grid=(N // BM,),
        in_specs=[_core_spec(D_H, 0), _core_spec(D_H, 1), _row_spec(D_H),
                  _full_spec(D_H, D_H), _full_spec(1, D_H),
                  _core_spec(DW, 0), _core_spec(DW, 1)],
        out_specs=_row_spec(D_H),
        out_shape=jax.ShapeDtypeStruct((N, D_H), jnp.float32),
    )(a, a, g, w, b, degp, degp)


def _tc_fin(a, g, b, degp):
    return pl.pallas_call(
        _fin_body,
        grid=(N // BM,),
        in_specs=[_core_spec(D_H, 0), _core_spec(D_H, 1), _row_spec(D_H),
                  _full_spec(1, D_OUT), _core_spec(DW, 0), _core_spec(DW, 1)],
        out_specs=_row_spec(D_OUT),
        out_shape=jax.ShapeDtypeStruct((N, D_OUT), jnp.float32),
    )(a, a, g, b, degp, degp)


def kernel(x, edge_index, W0, b0, W1, b1, W2, b2):
    edge4 = edge_index.astype(jnp.int32).reshape(2, NT, NCH, CB)
    ones = jnp.ones((CB, DW), jnp.float32)
    zeros16 = jnp.zeros((NPAD, DW), jnp.float32)
    zeros64 = jnp.zeros((NPAD, D_H), jnp.float32)

    degp = _sc_degree(edge4, ones, zeros16)
    h0 = _tc_mm(x, W0)

    g0 = _tc_pre(h0, degp)
    a0 = _sc_aggregate(edge4, g0, zeros64)
    g1 = _tc_mid(a0, g0, W1, b0.reshape(1, D_H), degp)
    a1 = _sc_aggregate(edge4, g1, zeros64)
    g2 = _tc_mid(a1, g1, W2, b1.reshape(1, D_H), degp)
    a2 = _sc_aggregate(edge4, g2, zeros64)
    out = _tc_fin(a2, g2, b2.reshape(1, D_OUT), degp)
    return out


# speedup vs baseline: 1.0230x; 1.0230x over previous
import jax
import jax.numpy as jnp
from jax import lax
from jax.experimental import pallas as pl
from jax.experimental.pallas import tpu as pltpu
from jax.experimental.pallas import tpu_sc as plsc

N = 10000
NPAD = 10240
E = 320000
D_IN = 128
D_H = 64
D_OUT = 64
NC = 2
NS = 16
NT = NC * NS
CB = 80
NCH = 125
EPT = CB * NCH
E_PAD = NT * EPT
DW = 16
RPT = NPAD // NS
NBUF = 6
BM = 2000

_mesh = plsc.VectorSubcoreMesh(core_axis_name="c", subcore_axis_name="s",
                               num_cores=NC, num_subcores=NS)
_sc_params = pltpu.CompilerParams(use_tc_tiling_on_sc=False)


def _deg_body(edge_hbm, ones_hbm, zeros_hbm, out_hbm, idx_v, ones_v, acc_sh, sem):
    c = lax.axis_index("c")
    s = lax.axis_index("s")
    wid = s * NC + c
    pltpu.sync_copy(zeros_hbm.at[pl.ds(s * RPT, RPT)], acc_sh.at[pl.ds(s * RPT, RPT)])
    pltpu.sync_copy(edge_hbm.at[1, wid], idx_v)
    pltpu.sync_copy(ones_hbm, ones_v)
    plsc.subcore_barrier()

    @pl.loop(0, NCH)
    def _(j):
        pltpu.sync_copy(ones_v, acc_sh.at[idx_v.at[j]], add=True)

    plsc.subcore_barrier()
    pltpu.sync_copy(acc_sh.at[pl.ds(s * RPT, RPT)],
                    out_hbm.at[c, pl.ds(s * RPT, RPT)])


def _sc_degree(edge4, ones, zeros16):
    f = pl.kernel(
        _deg_body,
        out_type=jax.ShapeDtypeStruct((NC, NPAD, DW), jnp.float32),
        mesh=_mesh,
        scratch_types=[
            pltpu.VMEM((NCH, CB), jnp.int32),
            pltpu.VMEM((CB, DW), jnp.float32),
            pltpu.VMEM_SHARED((NPAD, DW), jnp.float32),
            pltpu.SemaphoreType.DMA,
        ],
        compiler_params=_sc_params,
    )
    return f(edge4, ones, zeros16)


def _scat_body(edge_hbm, g_hbm, zeros_hbm, out_hbm,
               si_v, di_v, bufs, acc_sh, gsems):
    c = lax.axis_index("c")
    s = lax.axis_index("s")
    wid = s * NC + c
    pltpu.sync_copy(zeros_hbm.at[pl.ds(s * RPT, RPT)], acc_sh.at[pl.ds(s * RPT, RPT)])
    pltpu.sync_copy(edge_hbm.at[0, wid], si_v)
    pltpu.sync_copy(edge_hbm.at[1, wid], di_v)
    plsc.subcore_barrier()

    def fire_gather(j, r):
        pltpu.async_copy(g_hbm.at[si_v.at[j]], bufs[r], gsems[r])

    def wait_gather(r):
        pltpu.make_async_copy(g_hbm.at[si_v.at[0]], bufs[r], gsems[r]).wait()

    for k in range(NBUF):
        fire_gather(k, k)

    steady = NCH - NBUF
    @pl.loop(0, steady // NBUF)
    def _(i):
        for k in range(NBUF):
            t = i * NBUF + k
            wait_gather(k)
            pltpu.sync_copy(bufs[k], acc_sh.at[di_v.at[t]], add=True)
            fire_gather(t + NBUF, k)

    for t in range(NBUF * (steady // NBUF), NCH):
        k = t % NBUF
        wait_gather(k)
        pltpu.sync_copy(bufs[k], acc_sh.at[di_v.at[t]], add=True)
        if t + NBUF < NCH:
            fire_gather(t + NBUF, k)

    plsc.subcore_barrier()
    pltpu.sync_copy(acc_sh.at[pl.ds(s * RPT, RPT)],
                    out_hbm.at[c, pl.ds(s * RPT, RPT)])


def _sc_aggregate(edge4, g, zeros64):
    f = pl.kernel(
        _scat_body,
        out_type=jax.ShapeDtypeStruct((NC, NPAD, D_H), jnp.float32),
        mesh=_mesh,
        scratch_types=[
            pltpu.VMEM((NCH, CB), jnp.int32),
            pltpu.VMEM((NCH, CB), jnp.int32),
            [pltpu.VMEM((CB, D_H), jnp.float32) for _ in range(NBUF)],
            pltpu.VMEM_SHARED((NPAD, D_H), jnp.float32),
            [pltpu.SemaphoreType.DMA for _ in range(NBUF)],
        ],
        compiler_params=_sc_params,
    )
    return f(edge4, g, zeros64)


def _dis_block(dga_ref, dgb_ref):
    deg = jnp.sum(dga_ref[0] + dgb_ref[0], axis=1, keepdims=True) * (1.0 / DW) + 1.0
    return 1.0 / jnp.sqrt(deg)


def _pre_body(x_ref, w_ref, dga_ref, dgb_ref, g_ref):
    dis = _dis_block(dga_ref, dgb_ref)
    h = jnp.dot(x_ref[...], w_ref[...], preferred_element_type=jnp.float32)
    g_ref[...] = h * dis


def _mid_body(aa_ref, ab_ref, g_ref, w_ref, b_ref, dga_ref, dgb_ref, o_ref):
    dis = _dis_block(dga_ref, dgb_ref)
    act = dis * (aa_ref[0] + ab_ref[0] + g_ref[...]) + b_ref[...]
    act = jnp.maximum(act, 0.0)
    h = jnp.dot(act, w_ref[...], preferred_element_type=jnp.float32)
    o_ref[...] = h * dis


def _fin_body(aa_ref, ab_ref, g_ref, b_ref, dga_ref, dgb_ref, o_ref):
    dis = _dis_block(dga_ref, dgb_ref)
    o_ref[...] = dis * (aa_ref[0] + ab_ref[0] + g_ref[...]) + b_ref[...]


def _row_spec(w):
    return pl.BlockSpec((BM, w), lambda i: (i, 0))


def _core_spec(w, core):
    return pl.BlockSpec((1, BM, w), lambda i, _c=core: (_c, i, 0))


def _full_spec(h, w):
    return pl.BlockSpec((h, w), lambda i: (0, 0))


def _tc_pre(x, w0, degp):
    return pl.pallas_call(
        _pre_body,
        grid=(N // BM,),
        in_specs=[_row_spec(D_IN), _full_spec(D_IN, D_H),
                  _core_spec(DW, 0), _core_spec(DW, 1)],
        out_specs=_row_spec(D_H),
        out_shape=jax.ShapeDtypeStruct((N, D_H), jnp.float32),
    )(x, w0, degp, degp)


def _tc_mid(a, g, w, b, degp):
    return pl.pallas_call(
        _mid_body,
        grid=(N // BM,),
        in_specs=[_core_spec(D_H, 0), _core_spec(D_H, 1), _row_spec(D_H),
                  _full_spec(D_H, D_H), _full_spec(1, D_H),
                  _core_spec(DW, 0), _core_spec(DW, 1)],
        out_specs=_row_spec(D_H),
        out_shape=jax.ShapeDtypeStruct((N, D_H), jnp.float32),
    )(a, a, g, w, b, degp, degp)


def _tc_fin(a, g, b, degp):
    return pl.pallas_call(
        _fin_body,
        grid=(N // BM,),
        in_specs=[_core_spec(D_H, 0), _core_spec(D_H, 1), _row_spec(D_H),
                  _full_spec(1, D_OUT), _core_spec(DW, 0), _core_spec(DW, 1)],
        out_specs=_row_spec(D_OUT),
        out_shape=jax.ShapeDtypeStruct((N, D_OUT), jnp.float32),
    )(a, a, g, b, degp, degp)


def kernel(x, edge_index, W0, b0, W1, b1, W2, b2):
    edge4 = edge_index.astype(jnp.int32).reshape(2, NT, NCH, CB)
    ones = jnp.ones((CB, DW), jnp.float32)
    zeros16 = jnp.zeros((NPAD, DW), jnp.float32)
    zeros64 = jnp.zeros((NPAD, D_H), jnp.float32)

    degp = _sc_degree(edge4, ones, zeros16)

    g0 = _tc_pre(x, W0, degp)
    a0 = _sc_aggregate(edge4, g0, zeros64)
    g1 = _tc_mid(a0, g0, W1, b0.reshape(1, D_H), degp)
    a1 = _sc_aggregate(edge4, g1, zeros64)
    g2 = _tc_mid(a1, g1, W2, b1.reshape(1, D_H), degp)
    a2 = _sc_aggregate(edge4, g2, zeros64)
    out = _tc_fin(a2, g2, b2.reshape(1, D_OUT), degp)
    return out
